# baseline (speedup 1.0000x reference)
import functools

import jax
import jax.numpy as jnp
from jax import lax
from jax.experimental import pallas as pl
from jax.experimental.pallas import tpu as pltpu
from jax.experimental.pallas import tpu_sc as plsc

VOCAB = 1000000
DIM = 64
L = 200
MARGIN = 1.0

NS = 16
PAIRS_PER_W = 16
P = NS * PAIRS_PER_W


def _sc_body(e_hbm, idx_hbm, u_hbm, out_hbm,
             idx_v, rows_v, u_v, loss_v, stage_v, shared, sem):
    wid = lax.axis_index("s")
    base = wid * (2 * PAIRS_PER_W)

    pltpu.sync_copy(idx_hbm.at[pl.ds(base, 2 * PAIRS_PER_W)], idx_v)
    pltpu.sync_copy(u_hbm, u_v)
    pltpu.async_copy(e_hbm.at[idx_v], rows_v, sem).wait()

    partial = jnp.float32(0.0)
    for j in range(PAIRS_PER_W):
        acc = ((rows_v[j, pl.ds(0, 16)] - rows_v[j + PAIRS_PER_W, pl.ds(0, 16)])
               * u_v[pl.ds(0, 16)])
        for c in range(1, DIM // 16):
            acc = acc + ((rows_v[j, pl.ds(c * 16, 16)]
                          - rows_v[j + PAIRS_PER_W, pl.ds(c * 16, 16)])
                         * u_v[pl.ds(c * 16, 16)])
        s = jnp.sum(acc)
        hinge = jnp.maximum(jnp.float32(0.0), MARGIN - s)
        valid = (wid * PAIRS_PER_W + j) < L
        partial = partial + jnp.where(valid, hinge, jnp.float32(0.0))

    loss_v[...] = jnp.full((16,), partial, jnp.float32)
    pltpu.sync_copy(loss_v, shared.at[wid])
    plsc.subcore_barrier()

    @pl.when(wid == 0)
    def _():
        pltpu.sync_copy(shared, stage_v)
        tot = stage_v[0]
        for w in range(1, NS):
            tot = tot + stage_v[w]
        mean = jnp.sum(tot) * (1.0 / (16.0 * L))
        loss_v[...] = jnp.full((16,), mean, jnp.float32)
        pltpu.sync_copy(loss_v, out_hbm)


_sc_call = functools.partial(
    pl.kernel,
    out_type=jax.ShapeDtypeStruct((16,), jnp.float32),
    mesh=plsc.VectorSubcoreMesh(core_axis_name="c", subcore_axis_name="s",
                                num_cores=1, num_subcores=NS),
    compiler_params=pltpu.CompilerParams(needs_layout_passes=False,
                                         use_tc_tiling_on_sc=False),
    scratch_types=[
        pltpu.VMEM((2 * PAIRS_PER_W,), jnp.int32),
        pltpu.VMEM((2 * PAIRS_PER_W, DIM), jnp.float32),
        pltpu.VMEM((DIM,), jnp.float32),
        pltpu.VMEM((16,), jnp.float32),
        pltpu.VMEM((NS, 16), jnp.float32),
        pltpu.VMEM_SHARED((NS, 16), jnp.float32),
        pltpu.SemaphoreType.DMA,
    ],
)(_sc_body)


def kernel(pos_sample, neg_samples, E, U):
    pos = pos_sample.astype(jnp.int32)
    neg = neg_samples.astype(jnp.int32)
    posp = jnp.zeros((P,), jnp.int32).at[:L].set(pos)
    negp = jnp.zeros((P,), jnp.int32).at[:L].set(neg)
    idx = jnp.concatenate(
        [posp.reshape(NS, PAIRS_PER_W), negp.reshape(NS, PAIRS_PER_W)], axis=1
    ).reshape(2 * P)
    out = _sc_call(E, idx, U.reshape(DIM))
    return out[0]
